# Initial kernel scaffold; baseline (speedup 1.0000x reference)
import jax
import jax.numpy as jnp
from jax.experimental import pallas as pl


def kernel(x, edge_index, W1, b1, gamma1, beta1, W2, b2):
    raise NotImplementedError("write your pallas kernel here")


# speedup vs baseline: 21.0836x; 21.0836x over previous
import functools

import jax
import jax.numpy as jnp
from jax import lax
from jax.experimental import pallas as pl
from jax.experimental.pallas import tpu as pltpu
from jax.experimental.pallas import tpu_sc as plsc

NC = 2
NS = 16
NW = NC * NS
K = 80


def _sc_mesh():
    return plsc.VectorSubcoreMesh(core_axis_name="c", subcore_axis_name="s",
                                  num_cores=NC, num_subcores=NS)


def _make_degree_kernel(npad, epw):
    @functools.partial(
        pl.kernel,
        out_type=jax.ShapeDtypeStruct((NW, npad), jnp.float32),
        mesh=_sc_mesh(),
        scratch_types=[
            pltpu.VMEM((epw,), jnp.int32),
            pltpu.VMEM((npad,), jnp.float32),
        ],
        compiler_params=pltpu.CompilerParams(needs_layout_passes=False),
    )
    def deg_kernel(dst_hbm, out_hbm, dstv, hist):
        c = lax.axis_index("c")
        s = lax.axis_index("s")
        wid = c * NS + s

        def zbody(j, _):
            hist[pl.ds(j * 16, 16)] = jnp.zeros((16,), jnp.float32)
            return 0
        lax.fori_loop(0, npad // 16, zbody, 0)

        pltpu.sync_copy(dst_hbm.at[wid], dstv)

        ones = jnp.ones((16,), jnp.float32)
        def body(j, _):
            idx = dstv[pl.ds(j * 16, 16)]
            plsc.addupdate_scatter(hist, [idx], ones)
            return 0
        lax.fori_loop(0, epw // 16, body, 0)

        pltpu.sync_copy(hist, out_hbm.at[wid])

    return deg_kernel


def _make_prop_kernel(npad, nb, d):
    rpt = npad // NS

    @functools.partial(
        pl.kernel,
        out_type=jax.ShapeDtypeStruct((NC, npad, d), jnp.float32),
        mesh=_sc_mesh(),
        scratch_types=[
            pltpu.VMEM((nb, K), jnp.int32),
            pltpu.VMEM((nb, K), jnp.int32),
            pltpu.VMEM((K, d), jnp.float32),
            pltpu.VMEM_SHARED((npad, d), jnp.float32),
            pltpu.SemaphoreType.DMA,
        ],
    )
    def prop_kernel(xs_hbm, src_hbm, dst_hbm, out_hbm, srcv, dstv, rows, acc, sem):
        c = lax.axis_index("c")
        s = lax.axis_index("s")
        wid = c * NS + s

        def zfill(r, _):
            rows[r, :] = jnp.zeros((d,), jnp.float32)
            return 0
        lax.fori_loop(0, K, zfill, 0)
        r0 = s * rpt
        def zbody(j, _):
            pltpu.sync_copy(rows, acc.at[pl.ds(r0 + j * K, K)])
            return 0
        lax.fori_loop(0, rpt // K, zbody, 0)
        plsc.subcore_barrier()

        pltpu.sync_copy(src_hbm.at[wid], srcv)
        pltpu.sync_copy(dst_hbm.at[wid], dstv)

        def body(i, _):
            pltpu.async_copy(xs_hbm.at[srcv.at[i]], rows, sem).wait()
            pltpu.sync_copy(rows, acc.at[dstv.at[i]], add=True)
            return 0
        lax.fori_loop(0, nb, body, 0)
        plsc.subcore_barrier()

        pltpu.sync_copy(acc.at[pl.ds(r0, rpt)], out_hbm.at[c, pl.ds(r0, rpt)])

    return prop_kernel


def _diag(dinv_row, blk):
    ir = lax.broadcasted_iota(jnp.int32, (blk, blk), 0)
    ic = lax.broadcasted_iota(jnp.int32, (blk, blk), 1)
    d = jnp.broadcast_to(dinv_row, (blk, blk))
    return jnp.where(ir == ic, d, 0.0)


def _make_tc_scale(n, npad, d_in, blk):
    def body(deg_ref, x_ref, dinv_ref, xs_ref):
        deg = jnp.sum(deg_ref[...], axis=0, keepdims=True) + 1.0
        dinv = lax.rsqrt(deg)
        dinv_ref[...] = dinv
        dmat = _diag(dinv, blk)
        xs_ref[...] = jnp.dot(dmat, x_ref[...],
                              preferred_element_type=jnp.float32)

    grid = (npad // blk,)
    return pl.pallas_call(
        body,
        grid=grid,
        in_specs=[
            pl.BlockSpec((NW, blk), lambda i: (0, i)),
            pl.BlockSpec((blk, d_in), lambda i: (i, 0)),
        ],
        out_specs=[
            pl.BlockSpec((1, blk), lambda i: (0, i)),
            pl.BlockSpec((blk, d_in), lambda i: (i, 0)),
        ],
        out_shape=[
            jax.ShapeDtypeStruct((1, npad), jnp.float32),
            jax.ShapeDtypeStruct((npad, d_in), jnp.float32),
        ],
    )


def _make_tc_mlp(npad, d_in, d_hid, d_out_p, blk):
    bn_c = float(1.0 / (1.0 + 1e-5) ** 0.5)

    def body(agga_ref, aggb_ref, xs_ref, dinv_ref, w1_ref, b1_ref, g1_ref,
             be1_ref, w2_ref, zs_ref):
        dmat = _diag(dinv_ref[...], blk)
        pre = jnp.dot(dmat, agga_ref[...] + aggb_ref[...] + xs_ref[...],
                      preferred_element_type=jnp.float32)
        h = jnp.dot(pre, w1_ref[...], preferred_element_type=jnp.float32)
        h = (h + b1_ref[...]) * (g1_ref[...] * bn_c) + be1_ref[...]
        h = jnp.maximum(h, 0.0)
        z = jnp.dot(h, w2_ref[...], preferred_element_type=jnp.float32)
        zs_ref[...] = jnp.dot(dmat, z, preferred_element_type=jnp.float32)

    grid = (npad // blk,)
    return pl.pallas_call(
        body,
        grid=grid,
        in_specs=[
            pl.BlockSpec((blk, d_in), lambda i: (i, 0)),
            pl.BlockSpec((blk, d_in), lambda i: (i, 0)),
            pl.BlockSpec((blk, d_in), lambda i: (i, 0)),
            pl.BlockSpec((1, blk), lambda i: (0, i)),
            pl.BlockSpec((d_in, d_hid), lambda i: (0, 0)),
            pl.BlockSpec((1, d_hid), lambda i: (0, 0)),
            pl.BlockSpec((1, d_hid), lambda i: (0, 0)),
            pl.BlockSpec((1, d_hid), lambda i: (0, 0)),
            pl.BlockSpec((d_hid, d_out_p), lambda i: (0, 0)),
        ],
        out_specs=pl.BlockSpec((blk, d_out_p), lambda i: (i, 0)),
        out_shape=jax.ShapeDtypeStruct((npad, d_out_p), jnp.float32),
    )


def _make_tc_final(npad, d_out_p, blk):
    def body(agga_ref, aggb_ref, zs_ref, dinv_ref, b2_ref, out_ref):
        dmat = _diag(dinv_ref[...], blk)
        out_ref[...] = jnp.dot(
            dmat, agga_ref[...] + aggb_ref[...] + zs_ref[...],
            preferred_element_type=jnp.float32) + b2_ref[...]

    grid = (npad // blk,)
    return pl.pallas_call(
        body,
        grid=grid,
        in_specs=[
            pl.BlockSpec((blk, d_out_p), lambda i: (i, 0)),
            pl.BlockSpec((blk, d_out_p), lambda i: (i, 0)),
            pl.BlockSpec((blk, d_out_p), lambda i: (i, 0)),
            pl.BlockSpec((1, blk), lambda i: (0, i)),
            pl.BlockSpec((1, d_out_p), lambda i: (0, 0)),
        ],
        out_specs=pl.BlockSpec((blk, d_out_p), lambda i: (i, 0)),
        out_shape=jax.ShapeDtypeStruct((npad, d_out_p), jnp.float32),
    )


def kernel(x, edge_index, W1, b1, gamma1, beta1, W2, b2):
    n, d_in = x.shape
    d_hid = W1.shape[1]
    d_out = W2.shape[1]
    d_out_p = 128
    e = edge_index.shape[1]
    assert e % (NW * K) == 0, "edge count must tile evenly"
    epw = e // NW
    nb = epw // K
    npad = 10240
    assert n <= npad and npad % (NS * K) == 0

    src = edge_index[0].reshape(NW, nb, K)
    dst = edge_index[1].reshape(NW, nb, K)
    dst_flat = edge_index[1].reshape(NW, epw)

    deg_pp = _make_degree_kernel(npad, epw)(dst_flat)

    blk = 1024
    x_pad = jnp.pad(x, ((0, npad - n), (0, 0)))
    dinv, xs = _make_tc_scale(n, npad, d_in, blk)(deg_pp, x_pad)

    agg1 = _make_prop_kernel(npad, nb, d_in)(xs, src, dst)

    w2p = jnp.pad(W2, ((0, 0), (0, d_out_p - d_out)))
    zs = _make_tc_mlp(npad, d_in, d_hid, d_out_p, blk)(
        agg1[0], agg1[1], xs, dinv,
        W1, b1.reshape(1, d_hid), gamma1.reshape(1, d_hid),
        beta1.reshape(1, d_hid), w2p)

    agg2 = _make_prop_kernel(npad, nb, d_out_p)(zs, src, dst)

    b2p = jnp.pad(b2, (0, d_out_p - d_out)).reshape(1, d_out_p)
    outp = _make_tc_final(npad, d_out_p, blk)(
        agg2[0], agg2[1], zs, dinv, b2p)

    return outp[:n, :d_out]


# speedup vs baseline: 33.9019x; 1.6080x over previous
import functools

import jax
import jax.numpy as jnp
from jax import lax
from jax.experimental import pallas as pl
from jax.experimental.pallas import tpu as pltpu
from jax.experimental.pallas import tpu_sc as plsc

NC = 2
NS = 16
NW = NC * NS
K = 125


def _sc_mesh():
    return plsc.VectorSubcoreMesh(core_axis_name="c", subcore_axis_name="s",
                                  num_cores=NC, num_subcores=NS)


def _make_degree_kernel(npad, epw):
    @functools.partial(
        pl.kernel,
        out_type=jax.ShapeDtypeStruct((NW, npad), jnp.float32),
        mesh=_sc_mesh(),
        scratch_types=[
            pltpu.VMEM((epw,), jnp.int32),
            pltpu.VMEM((npad,), jnp.float32),
        ],
        compiler_params=pltpu.CompilerParams(needs_layout_passes=False),
    )
    def deg_kernel(dst_hbm, out_hbm, dstv, hist):
        c = lax.axis_index("c")
        s = lax.axis_index("s")
        wid = c * NS + s

        def zbody(j, _):
            hist[pl.ds(j * 16, 16)] = jnp.zeros((16,), jnp.float32)
            return 0
        lax.fori_loop(0, npad // 16, zbody, 0)

        pltpu.sync_copy(dst_hbm.at[wid], dstv)

        ones = jnp.ones((16,), jnp.float32)
        def body(j, _):
            idx = dstv[pl.ds(j * 16, 16)]
            plsc.addupdate_scatter(hist, [idx], ones)
            return 0
        lax.fori_loop(0, epw // 16, body, 0)

        pltpu.sync_copy(hist, out_hbm.at[wid])

    return deg_kernel


CB = 4


def _make_prop_kernel(npad, nch, d):
    rpt = npad // NS
    zc = 80
    assert nch % 2 == 0 and nch >= 4 and rpt % zc == 0 and zc <= K

    @functools.partial(
        pl.kernel,
        out_type=jax.ShapeDtypeStruct((NC, npad, d), jnp.float32),
        mesh=_sc_mesh(),
        scratch_types=[
            pltpu.VMEM((2, CB, K), jnp.int32),
            pltpu.VMEM((2, CB, K), jnp.int32),
            pltpu.VMEM((K, d), jnp.float32),
            pltpu.VMEM((K, d), jnp.float32),
            pltpu.VMEM_SHARED((npad, d), jnp.float32),
            pltpu.SemaphoreType.DMA,
            pltpu.SemaphoreType.DMA,
            pltpu.SemaphoreType.DMA,
            pltpu.SemaphoreType.DMA,
        ],
    )
    def prop_kernel(xs_hbm, src_hbm, dst_hbm, out_hbm, sidx, didx, rows_a,
                    rows_b, acc, sem_a, sem_b, sem_i0, sem_i1):
        c = lax.axis_index("c")
        s = lax.axis_index("s")
        wid = c * NS + s
        rows = (rows_a, rows_b)
        gsem = (sem_a, sem_b)
        isem = (sem_i0, sem_i1)

        def zfill(r, _):
            rows_a[r, :] = jnp.zeros((d,), jnp.float32)
            return 0
        lax.fori_loop(0, zc, zfill, 0)
        r0 = s * rpt
        def zbody(j, _):
            pltpu.sync_copy(rows_a.at[pl.ds(0, zc)],
                            acc.at[pl.ds(r0 + j * zc, zc)])
            return 0
        lax.fori_loop(0, rpt // zc, zbody, 0)
        plsc.subcore_barrier()

        def stage(q, slot):
            pltpu.async_copy(src_hbm.at[wid, q], sidx.at[slot], isem[slot])
            pltpu.async_copy(dst_hbm.at[wid, q], didx.at[slot], isem[slot])

        def stage_wait(slot):
            pltpu.make_async_copy(src_hbm.at[wid, 0], sidx.at[slot],
                                  isem[slot]).wait()
            pltpu.make_async_copy(dst_hbm.at[wid, 0], didx.at[slot],
                                  isem[slot]).wait()

        def gather(slot, b, rslot):
            pltpu.async_copy(xs_hbm.at[sidx.at[slot, b]], rows[rslot],
                             gsem[rslot])

        def gather_wait(rslot):
            pltpu.make_async_copy(xs_hbm.at[sidx.at[0, 0]], rows[rslot],
                                  gsem[rslot]).wait()

        def scatter(slot, b, rslot):
            pltpu.sync_copy(rows[rslot], acc.at[didx.at[slot, b]], add=True)

        stage(0, 0)
        stage(1, 1)
        stage_wait(0)
        gather(0, 0, 0)

        def body(cp, _):
            for rel in range(2 * CB):
                slot, b = divmod(rel, CB)
                nslot, nb_ = divmod(rel + 1, CB)
                if rel == CB - 1:
                    stage_wait(1)
                if rel == 2 * CB - 1:
                    stage_wait(0)
                gather(nslot % 2, nb_ % CB, (rel + 1) % 2)
                gather_wait(rel % 2)
                scatter(slot, b, rel % 2)
                if rel == CB - 1:
                    stage(lax.rem(2 * cp + 2, nch), 0)
                if rel == 2 * CB - 1:
                    stage(lax.rem(2 * cp + 3, nch), 1)
            return 0
        lax.fori_loop(0, nch // 2, body, 0)

        gather_wait(0)
        stage_wait(1)
        plsc.subcore_barrier()

        pltpu.sync_copy(acc.at[pl.ds(r0, rpt)], out_hbm.at[c, pl.ds(r0, rpt)])

    return prop_kernel


def _diag(dinv_row, blk):
    ir = lax.broadcasted_iota(jnp.int32, (blk, blk), 0)
    ic = lax.broadcasted_iota(jnp.int32, (blk, blk), 1)
    d = jnp.broadcast_to(dinv_row, (blk, blk))
    return jnp.where(ir == ic, d, 0.0)


def _make_tc_scale(n, npad, d_in, blk):
    def body(deg_ref, x_ref, dinv_ref, xs_ref):
        deg = jnp.sum(deg_ref[...], axis=0, keepdims=True) + 1.0
        dinv = lax.rsqrt(deg)
        dinv_ref[...] = dinv
        dmat = _diag(dinv, blk)
        xs_ref[...] = jnp.dot(dmat, x_ref[...],
                              preferred_element_type=jnp.float32)

    grid = (npad // blk,)
    return pl.pallas_call(
        body,
        grid=grid,
        in_specs=[
            pl.BlockSpec((NW, blk), lambda i: (0, i)),
            pl.BlockSpec((blk, d_in), lambda i: (i, 0)),
        ],
        out_specs=[
            pl.BlockSpec((1, blk), lambda i: (0, i)),
            pl.BlockSpec((blk, d_in), lambda i: (i, 0)),
        ],
        out_shape=[
            jax.ShapeDtypeStruct((1, npad), jnp.float32),
            jax.ShapeDtypeStruct((npad, d_in), jnp.float32),
        ],
    )


def _make_tc_mlp(npad, d_in, d_hid, d_out_p, blk):
    bn_c = float(1.0 / (1.0 + 1e-5) ** 0.5)

    def body(agga_ref, aggb_ref, xs_ref, dinv_ref, w1_ref, b1_ref, g1_ref,
             be1_ref, w2_ref, zs_ref):
        dmat = _diag(dinv_ref[...], blk)
        pre = jnp.dot(dmat, agga_ref[...] + aggb_ref[...] + xs_ref[...],
                      preferred_element_type=jnp.float32)
        h = jnp.dot(pre, w1_ref[...], preferred_element_type=jnp.float32)
        h = (h + b1_ref[...]) * (g1_ref[...] * bn_c) + be1_ref[...]
        h = jnp.maximum(h, 0.0)
        z = jnp.dot(h, w2_ref[...], preferred_element_type=jnp.float32)
        zs_ref[...] = jnp.dot(dmat, z, preferred_element_type=jnp.float32)

    grid = (npad // blk,)
    return pl.pallas_call(
        body,
        grid=grid,
        in_specs=[
            pl.BlockSpec((blk, d_in), lambda i: (i, 0)),
            pl.BlockSpec((blk, d_in), lambda i: (i, 0)),
            pl.BlockSpec((blk, d_in), lambda i: (i, 0)),
            pl.BlockSpec((1, blk), lambda i: (0, i)),
            pl.BlockSpec((d_in, d_hid), lambda i: (0, 0)),
            pl.BlockSpec((1, d_hid), lambda i: (0, 0)),
            pl.BlockSpec((1, d_hid), lambda i: (0, 0)),
            pl.BlockSpec((1, d_hid), lambda i: (0, 0)),
            pl.BlockSpec((d_hid, d_out_p), lambda i: (0, 0)),
        ],
        out_specs=pl.BlockSpec((blk, d_out_p), lambda i: (i, 0)),
        out_shape=jax.ShapeDtypeStruct((npad, d_out_p), jnp.float32),
    )


def _make_tc_final(npad, d_out_p, blk):
    def body(agga_ref, aggb_ref, zs_ref, dinv_ref, b2_ref, out_ref):
        dmat = _diag(dinv_ref[...], blk)
        out_ref[...] = jnp.dot(
            dmat, agga_ref[...] + aggb_ref[...] + zs_ref[...],
            preferred_element_type=jnp.float32) + b2_ref[...]

    grid = (npad // blk,)
    return pl.pallas_call(
        body,
        grid=grid,
        in_specs=[
            pl.BlockSpec((blk, d_out_p), lambda i: (i, 0)),
            pl.BlockSpec((blk, d_out_p), lambda i: (i, 0)),
            pl.BlockSpec((blk, d_out_p), lambda i: (i, 0)),
            pl.BlockSpec((1, blk), lambda i: (0, i)),
            pl.BlockSpec((1, d_out_p), lambda i: (0, 0)),
        ],
        out_specs=pl.BlockSpec((blk, d_out_p), lambda i: (i, 0)),
        out_shape=jax.ShapeDtypeStruct((npad, d_out_p), jnp.float32),
    )


def kernel(x, edge_index, W1, b1, gamma1, beta1, W2, b2):
    n, d_in = x.shape
    d_hid = W1.shape[1]
    d_out = W2.shape[1]
    d_out_p = 128
    e = edge_index.shape[1]
    assert e % (NW * CB * K) == 0, "edge count must tile evenly"
    epw = e // NW
    nch = epw // (CB * K)
    npad = 10240
    assert n <= npad

    src = edge_index[0].reshape(NW, nch, CB, K)
    dst = edge_index[1].reshape(NW, nch, CB, K)
    dst_flat = edge_index[1].reshape(NW, epw)

    deg_pp = _make_degree_kernel(npad, epw)(dst_flat)

    blk = 1024
    x_pad = jnp.pad(x, ((0, npad - n), (0, 0)))
    dinv, xs = _make_tc_scale(n, npad, d_in, blk)(deg_pp, x_pad)

    agg1 = _make_prop_kernel(npad, nch, d_in)(xs, src, dst)

    w2p = jnp.pad(W2, ((0, 0), (0, d_out_p - d_out)))
    zs = _make_tc_mlp(npad, d_in, d_hid, d_out_p, blk)(
        agg1[0], agg1[1], xs, dinv,
        W1, b1.reshape(1, d_hid), gamma1.reshape(1, d_hid),
        beta1.reshape(1, d_hid), w2p)

    agg2 = _make_prop_kernel(npad, nch, d_out_p)(zs, src, dst)

    b2p = jnp.pad(b2, (0, d_out_p - d_out)).reshape(1, d_out_p)
    outp = _make_tc_final(npad, d_out_p, blk)(
        agg2[0], agg2[1], zs, dinv, b2p)

    return outp[:n, :d_out]


# speedup vs baseline: 36.6597x; 1.0813x over previous
import functools

import jax
import jax.numpy as jnp
from jax import lax
from jax.experimental import pallas as pl
from jax.experimental.pallas import tpu as pltpu
from jax.experimental.pallas import tpu_sc as plsc

NC = 2
NS = 16
NW = NC * NS
K = 125


def _sc_mesh():
    return plsc.VectorSubcoreMesh(core_axis_name="c", subcore_axis_name="s",
                                  num_cores=NC, num_subcores=NS)


def _make_degree_kernel(npad, epw):
    @functools.partial(
        pl.kernel,
        out_type=jax.ShapeDtypeStruct((NW, npad), jnp.float32),
        mesh=_sc_mesh(),
        scratch_types=[
            pltpu.VMEM((epw,), jnp.int32),
            pltpu.VMEM((npad,), jnp.float32),
        ],
        compiler_params=pltpu.CompilerParams(needs_layout_passes=False),
    )
    def deg_kernel(dst_hbm, out_hbm, dstv, hist):
        c = lax.axis_index("c")
        s = lax.axis_index("s")
        wid = c * NS + s

        def zbody(j, _):
            hist[pl.ds(j * 16, 16)] = jnp.zeros((16,), jnp.float32)
            return 0
        lax.fori_loop(0, npad // 16, zbody, 0)

        pltpu.sync_copy(dst_hbm.at[wid], dstv)

        ones = jnp.ones((16,), jnp.float32)
        def body(j, _):
            idx = dstv[pl.ds(j * 16, 16)]
            plsc.addupdate_scatter(hist, [idx], ones)
            return 0
        lax.fori_loop(0, epw // 16, body, 0)

        pltpu.sync_copy(hist, out_hbm.at[wid])

    return deg_kernel


CB = 4


def _make_prop_kernel(npad, nch, d, tc_tiling=None):
    rpt = npad // NS
    zc = 80
    assert nch % 2 == 0 and nch >= 4 and rpt % zc == 0 and zc <= K

    @functools.partial(
        pl.kernel,
        out_type=jax.ShapeDtypeStruct((NC, npad, d), jnp.float32),
        mesh=_sc_mesh(),
        scratch_types=[
            pltpu.VMEM((2, CB, K), jnp.int32),
            pltpu.VMEM((2, CB, K), jnp.int32),
            pltpu.VMEM((K, d), jnp.float32),
            pltpu.VMEM((K, d), jnp.float32),
            pltpu.VMEM_SHARED((npad, d), jnp.float32),
            pltpu.SemaphoreType.DMA,
            pltpu.SemaphoreType.DMA,
            pltpu.SemaphoreType.DMA,
            pltpu.SemaphoreType.DMA,
        ],
        compiler_params=pltpu.CompilerParams(use_tc_tiling_on_sc=tc_tiling),
    )
    def prop_kernel(xs_hbm, src_hbm, dst_hbm, out_hbm, sidx, didx, rows_a,
                    rows_b, acc, sem_a, sem_b, sem_i0, sem_i1):
        c = lax.axis_index("c")
        s = lax.axis_index("s")
        wid = c * NS + s
        rows = (rows_a, rows_b)
        gsem = (sem_a, sem_b)
        isem = (sem_i0, sem_i1)

        def zfill(r, _):
            rows_a[r, :] = jnp.zeros((d,), jnp.float32)
            return 0
        lax.fori_loop(0, zc, zfill, 0)
        r0 = s * rpt
        def zbody(j, _):
            pltpu.sync_copy(rows_a.at[pl.ds(0, zc)],
                            acc.at[pl.ds(r0 + j * zc, zc)])
            return 0
        lax.fori_loop(0, rpt // zc, zbody, 0)
        plsc.subcore_barrier()

        def stage(q, slot):
            pltpu.async_copy(src_hbm.at[wid, q], sidx.at[slot], isem[slot])
            pltpu.async_copy(dst_hbm.at[wid, q], didx.at[slot], isem[slot])

        def stage_wait(slot):
            pltpu.make_async_copy(src_hbm.at[wid, 0], sidx.at[slot],
                                  isem[slot]).wait()
            pltpu.make_async_copy(dst_hbm.at[wid, 0], didx.at[slot],
                                  isem[slot]).wait()

        def gather(slot, b, rslot):
            pltpu.async_copy(xs_hbm.at[sidx.at[slot, b]], rows[rslot],
                             gsem[rslot])

        def gather_wait(rslot):
            pltpu.make_async_copy(xs_hbm.at[sidx.at[0, 0]], rows[rslot],
                                  gsem[rslot]).wait()

        def scatter(slot, b, rslot):
            pltpu.sync_copy(rows[rslot], acc.at[didx.at[slot, b]], add=True)

        stage(0, 0)
        stage(1, 1)
        stage_wait(0)
        gather(0, 0, 0)

        def body(cp, _):
            for rel in range(2 * CB):
                slot, b = divmod(rel, CB)
                nslot, nb_ = divmod(rel + 1, CB)
                if rel == CB - 1:
                    stage_wait(1)
                if rel == 2 * CB - 1:
                    stage_wait(0)
                gather(nslot % 2, nb_ % CB, (rel + 1) % 2)
                gather_wait(rel % 2)
                scatter(slot, b, rel % 2)
                if rel == CB - 1:
                    stage(lax.rem(2 * cp + 2, nch), 0)
                if rel == 2 * CB - 1:
                    stage(lax.rem(2 * cp + 3, nch), 1)
            return 0
        lax.fori_loop(0, nch // 2, body, 0)

        gather_wait(0)
        stage_wait(1)
        plsc.subcore_barrier()

        pltpu.sync_copy(acc.at[pl.ds(r0, rpt)], out_hbm.at[c, pl.ds(r0, rpt)])

    return prop_kernel


def _diag(dinv_row, blk):
    ir = lax.broadcasted_iota(jnp.int32, (blk, blk), 0)
    ic = lax.broadcasted_iota(jnp.int32, (blk, blk), 1)
    d = jnp.broadcast_to(dinv_row, (blk, blk))
    return jnp.where(ir == ic, d, 0.0)


def _make_tc_scale(n, npad, d_in, blk):
    def body(deg_ref, x_ref, dinv_ref, xs_ref):
        deg = jnp.sum(deg_ref[...], axis=0, keepdims=True) + 1.0
        dinv = lax.rsqrt(deg)
        dinv_ref[...] = dinv
        dmat = _diag(dinv, blk)
        xs_ref[...] = jnp.dot(dmat, x_ref[...],
                              preferred_element_type=jnp.float32)

    grid = (npad // blk,)
    return pl.pallas_call(
        body,
        grid=grid,
        in_specs=[
            pl.BlockSpec((NW, blk), lambda i: (0, i)),
            pl.BlockSpec((blk, d_in), lambda i: (i, 0)),
        ],
        out_specs=[
            pl.BlockSpec((1, blk), lambda i: (0, i)),
            pl.BlockSpec((blk, d_in), lambda i: (i, 0)),
        ],
        out_shape=[
            jax.ShapeDtypeStruct((1, npad), jnp.float32),
            jax.ShapeDtypeStruct((npad, d_in), jnp.float32),
        ],
    )


def _make_tc_mlp(npad, d_in, d_hid, d_out_p, blk):
    bn_c = float(1.0 / (1.0 + 1e-5) ** 0.5)

    def body(agga_ref, aggb_ref, xs_ref, dinv_ref, w1_ref, b1_ref, g1_ref,
             be1_ref, w2_ref, zs_ref):
        dmat = _diag(dinv_ref[...], blk)
        pre = jnp.dot(dmat, agga_ref[...] + aggb_ref[...] + xs_ref[...],
                      preferred_element_type=jnp.float32)
        h = jnp.dot(pre, w1_ref[...], preferred_element_type=jnp.float32)
        h = (h + b1_ref[...]) * (g1_ref[...] * bn_c) + be1_ref[...]
        h = jnp.maximum(h, 0.0)
        z = jnp.dot(h, w2_ref[...], preferred_element_type=jnp.float32)
        zs_ref[...] = jnp.dot(dmat, z, preferred_element_type=jnp.float32)

    grid = (npad // blk,)
    return pl.pallas_call(
        body,
        grid=grid,
        in_specs=[
            pl.BlockSpec((blk, d_in), lambda i: (i, 0)),
            pl.BlockSpec((blk, d_in), lambda i: (i, 0)),
            pl.BlockSpec((blk, d_in), lambda i: (i, 0)),
            pl.BlockSpec((1, blk), lambda i: (0, i)),
            pl.BlockSpec((d_in, d_hid), lambda i: (0, 0)),
            pl.BlockSpec((1, d_hid), lambda i: (0, 0)),
            pl.BlockSpec((1, d_hid), lambda i: (0, 0)),
            pl.BlockSpec((1, d_hid), lambda i: (0, 0)),
            pl.BlockSpec((d_hid, d_out_p), lambda i: (0, 0)),
        ],
        out_specs=pl.BlockSpec((blk, d_out_p), lambda i: (i, 0)),
        out_shape=jax.ShapeDtypeStruct((npad, d_out_p), jnp.float32),
    )


def _make_tc_final(npad, d_out_p, blk):
    def body(agga_ref, aggb_ref, zs_ref, dinv_ref, b2_ref, out_ref):
        dmat = _diag(dinv_ref[...], blk)
        out_ref[...] = jnp.dot(
            dmat, agga_ref[...] + aggb_ref[...] + zs_ref[...],
            preferred_element_type=jnp.float32) + b2_ref[...]

    grid = (npad // blk,)
    return pl.pallas_call(
        body,
        grid=grid,
        in_specs=[
            pl.BlockSpec((blk, d_out_p), lambda i: (i, 0)),
            pl.BlockSpec((blk, d_out_p), lambda i: (i, 0)),
            pl.BlockSpec((blk, d_out_p), lambda i: (i, 0)),
            pl.BlockSpec((1, blk), lambda i: (0, i)),
            pl.BlockSpec((1, d_out_p), lambda i: (0, 0)),
        ],
        out_specs=pl.BlockSpec((blk, d_out_p), lambda i: (i, 0)),
        out_shape=jax.ShapeDtypeStruct((npad, d_out_p), jnp.float32),
    )


def kernel(x, edge_index, W1, b1, gamma1, beta1, W2, b2):
    n, d_in = x.shape
    d_hid = W1.shape[1]
    d_out = W2.shape[1]
    d_out_p = 64
    e = edge_index.shape[1]
    assert e % (NW * CB * K) == 0, "edge count must tile evenly"
    epw = e // NW
    nch = epw // (CB * K)
    npad = 10240
    assert n <= npad

    src = edge_index[0].reshape(NW, nch, CB, K)
    dst = edge_index[1].reshape(NW, nch, CB, K)
    dst_flat = edge_index[1].reshape(NW, epw)

    deg_pp = _make_degree_kernel(npad, epw)(dst_flat)

    blk = 1024
    x_pad = jnp.pad(x, ((0, npad - n), (0, 0)))
    dinv, xs = _make_tc_scale(n, npad, d_in, blk)(deg_pp, x_pad)

    agg1 = _make_prop_kernel(npad, nch, d_in)(xs, src, dst)

    w2p = jnp.pad(W2, ((0, 0), (0, d_out_p - d_out)))
    zs = _make_tc_mlp(npad, d_in, d_hid, d_out_p, blk)(
        agg1[0], agg1[1], xs, dinv,
        W1, b1.reshape(1, d_hid), gamma1.reshape(1, d_hid),
        beta1.reshape(1, d_hid), w2p)

    agg2 = _make_prop_kernel(npad, nch, d_out_p, tc_tiling=False)(zs, src, dst)

    b2p = jnp.pad(b2, (0, d_out_p - d_out)).reshape(1, d_out_p)
    outp = _make_tc_final(npad, d_out_p, blk)(
        agg2[0], agg2[1], zs, dinv, b2p)

    return outp[:n, :d_out]


# speedup vs baseline: 38.3278x; 1.0455x over previous
import functools

import jax
import jax.numpy as jnp
from jax import lax
from jax.experimental import pallas as pl
from jax.experimental.pallas import tpu as pltpu
from jax.experimental.pallas import tpu_sc as plsc

NC = 2
NS = 16
NW = NC * NS
K = 125


def _sc_mesh():
    return plsc.VectorSubcoreMesh(core_axis_name="c", subcore_axis_name="s",
                                  num_cores=NC, num_subcores=NS)


def _make_degree_kernel(npad, epw):
    @functools.partial(
        pl.kernel,
        out_type=jax.ShapeDtypeStruct((NW, npad), jnp.float32),
        mesh=_sc_mesh(),
        scratch_types=[
            pltpu.VMEM((epw,), jnp.int32),
            pltpu.VMEM((npad,), jnp.float32),
        ],
        compiler_params=pltpu.CompilerParams(needs_layout_passes=False),
    )
    def deg_kernel(dst_hbm, out_hbm, dstv, hist):
        c = lax.axis_index("c")
        s = lax.axis_index("s")
        wid = c * NS + s

        def zbody(j, _):
            hist[pl.ds(j * 16, 16)] = jnp.zeros((16,), jnp.float32)
            return 0
        lax.fori_loop(0, npad // 16, zbody, 0)

        pltpu.sync_copy(dst_hbm.at[wid], dstv)

        ones = jnp.ones((16,), jnp.float32)
        def body(j, _):
            idx = dstv[pl.ds(j * 16, 16)]
            plsc.addupdate_scatter(hist, [idx], ones)
            return 0
        lax.fori_loop(0, epw // 16, body, 0)

        pltpu.sync_copy(hist, out_hbm.at[wid])

    return deg_kernel


CB = 4


def _make_prop_kernel(npad, nch, d, tc_tiling=None):
    rpt = npad // NS
    zc = 80
    assert nch % 2 == 0 and nch >= 4 and rpt % zc == 0 and zc <= K

    @functools.partial(
        pl.kernel,
        out_type=jax.ShapeDtypeStruct((NC, npad, d), jnp.float32),
        mesh=_sc_mesh(),
        scratch_types=[
            pltpu.VMEM((2, CB, K), jnp.int32),
            pltpu.VMEM((2, CB, K), jnp.int32),
            pltpu.VMEM((K, d), jnp.float32),
            pltpu.VMEM((K, d), jnp.float32),
            pltpu.VMEM_SHARED((npad, d), jnp.float32),
            pltpu.SemaphoreType.DMA,
            pltpu.SemaphoreType.DMA,
            pltpu.SemaphoreType.DMA,
            pltpu.SemaphoreType.DMA,
        ],
        compiler_params=pltpu.CompilerParams(use_tc_tiling_on_sc=tc_tiling),
    )
    def prop_kernel(xs_hbm, src_hbm, dst_hbm, out_hbm, sidx, didx, rows_a,
                    rows_b, acc, sem_a, sem_b, sem_i0, sem_i1):
        c = lax.axis_index("c")
        s = lax.axis_index("s")
        wid = c * NS + s
        rows = (rows_a, rows_b)
        gsem = (sem_a, sem_b)
        isem = (sem_i0, sem_i1)

        def zfill(r, _):
            rows_a[r, :] = jnp.zeros((d,), jnp.float32)
            return 0
        lax.fori_loop(0, zc, zfill, 0)
        r0 = s * rpt
        def zbody(j, _):
            pltpu.sync_copy(rows_a.at[pl.ds(0, zc)],
                            acc.at[pl.ds(r0 + j * zc, zc)])
            return 0
        lax.fori_loop(0, rpt // zc, zbody, 0)
        plsc.subcore_barrier()

        def stage(q, slot):
            pltpu.async_copy(src_hbm.at[wid, q], sidx.at[slot], isem[slot])
            pltpu.async_copy(dst_hbm.at[wid, q], didx.at[slot], isem[slot])

        def stage_wait(slot):
            pltpu.make_async_copy(src_hbm.at[wid, 0], sidx.at[slot],
                                  isem[slot]).wait()
            pltpu.make_async_copy(dst_hbm.at[wid, 0], didx.at[slot],
                                  isem[slot]).wait()

        def gather(slot, b, rslot):
            pltpu.async_copy(xs_hbm.at[sidx.at[slot, b]], rows[rslot],
                             gsem[rslot])

        def gather_wait(rslot):
            pltpu.make_async_copy(xs_hbm.at[sidx.at[0, 0]], rows[rslot],
                                  gsem[rslot]).wait()

        def scatter(slot, b, rslot):
            pltpu.sync_copy(rows[rslot], acc.at[didx.at[slot, b]], add=True)

        stage(0, 0)
        stage(1, 1)
        stage_wait(0)
        gather(0, 0, 0)

        def body(cp, _):
            for rel in range(2 * CB):
                slot, b = divmod(rel, CB)
                nslot, nb_ = divmod(rel + 1, CB)
                if rel == CB - 1:
                    stage_wait(1)
                if rel == 2 * CB - 1:
                    stage_wait(0)
                gather(nslot % 2, nb_ % CB, (rel + 1) % 2)
                gather_wait(rel % 2)
                scatter(slot, b, rel % 2)
                if rel == CB - 1:
                    stage(lax.rem(2 * cp + 2, nch), 0)
                if rel == 2 * CB - 1:
                    stage(lax.rem(2 * cp + 3, nch), 1)
            return 0
        lax.fori_loop(0, nch // 2, body, 0)

        gather_wait(0)
        stage_wait(1)
        plsc.subcore_barrier()

        pltpu.sync_copy(acc.at[pl.ds(r0, rpt)], out_hbm.at[c, pl.ds(r0, rpt)])

    return prop_kernel


def _diag(dinv_row, blk):
    ir = lax.broadcasted_iota(jnp.int32, (blk, blk), 0)
    ic = lax.broadcasted_iota(jnp.int32, (blk, blk), 1)
    d = jnp.broadcast_to(dinv_row, (blk, blk))
    return jnp.where(ir == ic, d, 0.0)


def _make_tc_scale(n, npad, d_in, blk):
    def body(deg_ref, x_ref, dinvc_ref, xs_ref):
        deg = jnp.sum(deg_ref[...], axis=0, keepdims=True) + 1.0
        dinv = lax.rsqrt(deg)
        dmat = _diag(dinv, blk)
        dinvc = jnp.dot(dmat, jnp.ones((blk, 8), jnp.float32),
                        preferred_element_type=jnp.float32)
        dinvc_ref[...] = dinvc
        xs_ref[...] = x_ref[...] * dinvc[:, 0:1]

    grid = (npad // blk,)
    return pl.pallas_call(
        body,
        grid=grid,
        in_specs=[
            pl.BlockSpec((NW, blk), lambda i: (0, i)),
            pl.BlockSpec((blk, d_in), lambda i: (i, 0)),
        ],
        out_specs=[
            pl.BlockSpec((blk, 8), lambda i: (i, 0)),
            pl.BlockSpec((blk, d_in), lambda i: (i, 0)),
        ],
        out_shape=[
            jax.ShapeDtypeStruct((npad, 8), jnp.float32),
            jax.ShapeDtypeStruct((npad, d_in), jnp.float32),
        ],
    )


def _make_tc_mlp(npad, d_in, d_hid, d_out_p, blk):
    bn_c = float(1.0 / (1.0 + 1e-5) ** 0.5)

    def body(agga_ref, aggb_ref, xs_ref, dinv_ref, w1_ref, b1_ref, g1_ref,
             be1_ref, w2_ref, zs_ref):
        dinvc = dinv_ref[:, 0:1]
        pre = (agga_ref[...] + aggb_ref[...] + xs_ref[...]) * dinvc
        h = jnp.dot(pre, w1_ref[...], preferred_element_type=jnp.float32)
        h = (h + b1_ref[...]) * (g1_ref[...] * bn_c) + be1_ref[...]
        h = jnp.maximum(h, 0.0)
        z = jnp.dot(h, w2_ref[...], preferred_element_type=jnp.float32)
        zs_ref[...] = z * dinvc

    grid = (npad // blk,)
    return pl.pallas_call(
        body,
        grid=grid,
        in_specs=[
            pl.BlockSpec((blk, d_in), lambda i: (i, 0)),
            pl.BlockSpec((blk, d_in), lambda i: (i, 0)),
            pl.BlockSpec((blk, d_in), lambda i: (i, 0)),
            pl.BlockSpec((blk, 8), lambda i: (i, 0)),
            pl.BlockSpec((d_in, d_hid), lambda i: (0, 0)),
            pl.BlockSpec((1, d_hid), lambda i: (0, 0)),
            pl.BlockSpec((1, d_hid), lambda i: (0, 0)),
            pl.BlockSpec((1, d_hid), lambda i: (0, 0)),
            pl.BlockSpec((d_hid, d_out_p), lambda i: (0, 0)),
        ],
        out_specs=pl.BlockSpec((blk, d_out_p), lambda i: (i, 0)),
        out_shape=jax.ShapeDtypeStruct((npad, d_out_p), jnp.float32),
    )


def _make_tc_final(npad, d_out_p, blk):
    def body(agga_ref, aggb_ref, zs_ref, dinv_ref, b2_ref, out_ref):
        out_ref[...] = (agga_ref[...] + aggb_ref[...] + zs_ref[...]) \
            * dinv_ref[:, 0:1] + b2_ref[...]

    grid = (npad // blk,)
    return pl.pallas_call(
        body,
        grid=grid,
        in_specs=[
            pl.BlockSpec((blk, d_out_p), lambda i: (i, 0)),
            pl.BlockSpec((blk, d_out_p), lambda i: (i, 0)),
            pl.BlockSpec((blk, d_out_p), lambda i: (i, 0)),
            pl.BlockSpec((blk, 8), lambda i: (i, 0)),
            pl.BlockSpec((1, d_out_p), lambda i: (0, 0)),
        ],
        out_specs=pl.BlockSpec((blk, d_out_p), lambda i: (i, 0)),
        out_shape=jax.ShapeDtypeStruct((npad, d_out_p), jnp.float32),
    )


def kernel(x, edge_index, W1, b1, gamma1, beta1, W2, b2):
    n, d_in = x.shape
    d_hid = W1.shape[1]
    d_out = W2.shape[1]
    d_out_p = 64
    e = edge_index.shape[1]
    assert e % (NW * CB * K) == 0, "edge count must tile evenly"
    epw = e // NW
    nch = epw // (CB * K)
    npad = 10240
    assert n <= npad

    src = edge_index[0].reshape(NW, nch, CB, K)
    dst = edge_index[1].reshape(NW, nch, CB, K)
    dst_flat = edge_index[1].reshape(NW, epw)

    deg_pp = _make_degree_kernel(npad, epw)(dst_flat)

    blk = 1024
    x_pad = jnp.pad(x, ((0, npad - n), (0, 0)))
    dinv, xs = _make_tc_scale(n, npad, d_in, blk)(deg_pp, x_pad)

    agg1 = _make_prop_kernel(npad, nch, d_in)(xs, src, dst)

    w2p = jnp.pad(W2, ((0, 0), (0, d_out_p - d_out)))
    zs = _make_tc_mlp(npad, d_in, d_hid, d_out_p, blk)(
        agg1[0], agg1[1], xs, dinv,
        W1, b1.reshape(1, d_hid), gamma1.reshape(1, d_hid),
        beta1.reshape(1, d_hid), w2p)

    agg2 = _make_prop_kernel(npad, nch, d_out_p, tc_tiling=False)(zs, src, dst)

    b2p = jnp.pad(b2, (0, d_out_p - d_out)).reshape(1, d_out_p)
    outp = _make_tc_final(npad, d_out_p, blk)(
        agg2[0], agg2[1], zs, dinv, b2p)

    return outp[:n, :d_out]


# speedup vs baseline: 41.9198x; 1.0937x over previous
import functools

import jax
import jax.numpy as jnp
from jax import lax
from jax.experimental import pallas as pl
from jax.experimental.pallas import tpu as pltpu
from jax.experimental.pallas import tpu_sc as plsc

NC = 2
NS = 16
NW = NC * NS
K = 128
CB = 6


def _sc_mesh():
    return plsc.VectorSubcoreMesh(core_axis_name="c", subcore_axis_name="s",
                                  num_cores=NC, num_subcores=NS)


def _row_span(wid, rpw, nx):
    return rpw * wid + jnp.minimum(wid, nx)


def _make_degree_kernel(npad, erows):
    rpw, nx = divmod(erows, NW)

    @functools.partial(
        pl.kernel,
        out_type=jax.ShapeDtypeStruct((NW, npad), jnp.float32),
        mesh=_sc_mesh(),
        scratch_types=[
            pltpu.VMEM((rpw + 1, K), jnp.int32),
            pltpu.VMEM((npad,), jnp.float32),
        ],
        compiler_params=pltpu.CompilerParams(needs_layout_passes=False,
                                             use_tc_tiling_on_sc=False),
    )
    def deg_kernel(ei_hbm, out_hbm, dstv, hist):
        c = lax.axis_index("c")
        s = lax.axis_index("s")
        wid = c * NS + s
        r0 = _row_span(wid, rpw, nx)

        def zbody(j, _):
            hist[pl.ds(j * 16, 16)] = jnp.zeros((16,), jnp.float32)
            return 0
        lax.fori_loop(0, npad // 16, zbody, 0)

        pltpu.sync_copy(ei_hbm.at[1, pl.ds(r0, rpw)], dstv.at[pl.ds(0, rpw)])

        ones = jnp.ones((16,), jnp.float32)
        def srow(r):
            for j in range(K // 16):
                plsc.addupdate_scatter(
                    hist, [dstv[r, pl.ds(j * 16, 16)]], ones)
        def body(r, _):
            srow(r)
            return 0
        lax.fori_loop(0, rpw, body, 0)
        @pl.when(wid < nx)
        def _():
            pltpu.sync_copy(ei_hbm.at[1, pl.ds(r0 + rpw, 1)],
                            dstv.at[pl.ds(rpw, 1)])
            srow(rpw)

        pltpu.sync_copy(hist, out_hbm.at[wid])

    return deg_kernel


def _make_prop_kernel(npad, erows, d, tc_tiling=None):
    rpw, nx = divmod(erows, NW)
    nch = rpw // CB
    assert rpw % CB == 0 and CB % 2 == 0 and nch >= 2
    rpt = npad // NS
    zc = 80
    assert rpt % zc == 0 and zc <= K

    @functools.partial(
        pl.kernel,
        out_type=jax.ShapeDtypeStruct((NC, npad, d), jnp.float32),
        mesh=_sc_mesh(),
        scratch_types=[
            pltpu.VMEM((2, CB, K), jnp.int32),
            pltpu.VMEM((2, CB, K), jnp.int32),
            pltpu.VMEM((K, d), jnp.float32),
            pltpu.VMEM((K, d), jnp.float32),
            pltpu.VMEM_SHARED((npad, d), jnp.float32),
            pltpu.SemaphoreType.DMA,
            pltpu.SemaphoreType.DMA,
            pltpu.SemaphoreType.DMA,
        ],
        compiler_params=pltpu.CompilerParams(use_tc_tiling_on_sc=tc_tiling),
    )
    def prop_kernel(xs_hbm, ei_hbm, out_hbm, sidx, didx, rows_a, rows_b,
                    acc, sem_a, sem_b, sem_i):
        c = lax.axis_index("c")
        s = lax.axis_index("s")
        wid = c * NS + s
        w0 = _row_span(wid, rpw, nx)
        rows = (rows_a, rows_b)
        gsem = (sem_a, sem_b)

        def zfill(r, _):
            rows_a[r, :] = jnp.zeros((d,), jnp.float32)
            return 0
        lax.fori_loop(0, zc, zfill, 0)
        a0 = s * rpt
        def zbody(j, _):
            pltpu.sync_copy(rows_a.at[pl.ds(0, zc)],
                            acc.at[pl.ds(a0 + j * zc, zc)])
            return 0
        lax.fori_loop(0, rpt // zc, zbody, 0)
        plsc.subcore_barrier()

        def stage(q, slot):
            r = w0 + q * CB
            pltpu.async_copy(ei_hbm.at[0, pl.ds(r, CB)], sidx.at[slot], sem_i)
            pltpu.async_copy(ei_hbm.at[1, pl.ds(r, CB)], didx.at[slot], sem_i)

        def stage_wait(slot):
            pltpu.make_async_copy(ei_hbm.at[0, pl.ds(0, CB)], sidx.at[slot],
                                  sem_i).wait()
            pltpu.make_async_copy(ei_hbm.at[0, pl.ds(0, CB)], didx.at[slot],
                                  sem_i).wait()

        def gather(slot, b, rslot):
            pltpu.async_copy(xs_hbm.at[sidx.at[slot, b]], rows[rslot],
                             gsem[rslot])

        def gather_wait(rslot):
            pltpu.make_async_copy(xs_hbm.at[sidx.at[0, 0]], rows[rslot],
                                  gsem[rslot]).wait()

        def scatter(slot, b, rslot):
            pltpu.sync_copy(rows[rslot], acc.at[didx.at[slot, b]], add=True)

        stage(0, 0)
        stage_wait(0)
        gather(0, 0, 0)

        def body(q, _):
            slot = lax.rem(q, 2)
            nslot = 1 - slot
            stage(lax.rem(q + 1, nch), nslot)
            for b in range(CB):
                if b == CB - 1:
                    stage_wait(nslot)
                    gather(nslot, 0, (b + 1) % 2)
                else:
                    gather(slot, b + 1, (b + 1) % 2)
                gather_wait(b % 2)
                scatter(slot, b, b % 2)
            return 0
        lax.fori_loop(0, nch, body, 0)
        gather_wait(0)

        @pl.when(wid < nx)
        def _():
            r = w0 + rpw
            pltpu.async_copy(ei_hbm.at[0, pl.ds(r, 1)],
                             sidx.at[0, pl.ds(0, 1)], sem_i)
            pltpu.async_copy(ei_hbm.at[1, pl.ds(r, 1)],
                             didx.at[0, pl.ds(0, 1)], sem_i)
            pltpu.make_async_copy(ei_hbm.at[0, pl.ds(0, 1)],
                                  sidx.at[0, pl.ds(0, 1)], sem_i).wait()
            pltpu.make_async_copy(ei_hbm.at[0, pl.ds(0, 1)],
                                  didx.at[0, pl.ds(0, 1)], sem_i).wait()
            pltpu.async_copy(xs_hbm.at[sidx.at[0, 0]], rows_a, sem_a)
            gather_wait(0)
            scatter(0, 0, 0)

        plsc.subcore_barrier()
        pltpu.sync_copy(acc.at[pl.ds(a0, rpt)], out_hbm.at[c, pl.ds(a0, rpt)])

    return prop_kernel


def _diag(dinv_row, blk):
    ir = lax.broadcasted_iota(jnp.int32, (blk, blk), 0)
    ic = lax.broadcasted_iota(jnp.int32, (blk, blk), 1)
    dm = jnp.broadcast_to(dinv_row, (blk, blk))
    return jnp.where(ir == ic, dm, 0.0)


def _make_tc_scale(n, npad, d_in, blk):
    def body(deg_ref, x_ref, dinvc_ref, xs_ref):
        deg = jnp.sum(deg_ref[...], axis=0, keepdims=True) + 1.0
        dinv = lax.rsqrt(deg)
        dmat = _diag(dinv, blk)
        dinvc = jnp.dot(dmat, jnp.ones((blk, 8), jnp.float32),
                        preferred_element_type=jnp.float32)
        dinvc_ref[...] = dinvc
        xs_ref[...] = x_ref[...] * dinvc[:, 0:1]

    grid = (npad // blk,)
    return pl.pallas_call(
        body,
        grid=grid,
        in_specs=[
            pl.BlockSpec((NW, blk), lambda i: (0, i)),
            pl.BlockSpec((blk, d_in), lambda i: (i, 0)),
        ],
        out_specs=[
            pl.BlockSpec((blk, 8), lambda i: (i, 0)),
            pl.BlockSpec((blk, d_in), lambda i: (i, 0)),
        ],
        out_shape=[
            jax.ShapeDtypeStruct((npad, 8), jnp.float32),
            jax.ShapeDtypeStruct((npad, d_in), jnp.float32),
        ],
    )


def _make_tc_mlp(npad, d_in, d_hid, d_out_p, blk):
    bn_c = float(1.0 / (1.0 + 1e-5) ** 0.5)

    def body(agg_ref, xs_ref, dinv_ref, w1_ref, b1_ref, g1_ref,
             be1_ref, w2_ref, zs_ref):
        dinvc = dinv_ref[:, 0:1]
        pre = (agg_ref[0] + agg_ref[1] + xs_ref[...]) * dinvc
        h = jnp.dot(pre, w1_ref[...], preferred_element_type=jnp.float32)
        h = (h + b1_ref[...]) * (g1_ref[...] * bn_c) + be1_ref[...]
        h = jnp.maximum(h, 0.0)
        z = jnp.dot(h, w2_ref[...], preferred_element_type=jnp.float32)
        zs_ref[...] = z * dinvc

    grid = (npad // blk,)
    return pl.pallas_call(
        body,
        grid=grid,
        in_specs=[
            pl.BlockSpec((2, blk, d_in), lambda i: (0, i, 0)),
            pl.BlockSpec((blk, d_in), lambda i: (i, 0)),
            pl.BlockSpec((blk, 8), lambda i: (i, 0)),
            pl.BlockSpec((d_in, d_hid), lambda i: (0, 0)),
            pl.BlockSpec((1, d_hid), lambda i: (0, 0)),
            pl.BlockSpec((1, d_hid), lambda i: (0, 0)),
            pl.BlockSpec((1, d_hid), lambda i: (0, 0)),
            pl.BlockSpec((d_hid, d_out_p), lambda i: (0, 0)),
        ],
        out_specs=pl.BlockSpec((blk, d_out_p), lambda i: (i, 0)),
        out_shape=jax.ShapeDtypeStruct((npad, d_out_p), jnp.float32),
    )


def _make_tc_final(n, npad, d_out_p, d_out, blk):
    def body(agg_ref, zs_ref, dinv_ref, b2_ref, out_ref):
        v = (agg_ref[0] + agg_ref[1] + zs_ref[...]) * dinv_ref[:, 0:1] \
            + b2_ref[...]
        out_ref[...] = v[:, :d_out]

    grid = (n // blk,)
    return pl.pallas_call(
        body,
        grid=grid,
        in_specs=[
            pl.BlockSpec((2, blk, d_out_p), lambda i: (0, i, 0)),
            pl.BlockSpec((blk, d_out_p), lambda i: (i, 0)),
            pl.BlockSpec((blk, 8), lambda i: (i, 0)),
            pl.BlockSpec((1, d_out_p), lambda i: (0, 0)),
        ],
        out_specs=pl.BlockSpec((blk, d_out), lambda i: (i, 0)),
        out_shape=jax.ShapeDtypeStruct((n, d_out), jnp.float32),
    )


def kernel(x, edge_index, W1, b1, gamma1, beta1, W2, b2):
    n, d_in = x.shape
    d_hid = W1.shape[1]
    d_out = W2.shape[1]
    d_out_p = 64
    e = edge_index.shape[1]
    assert e % K == 0
    erows = e // K
    npad = 10240
    assert n <= npad

    ei = edge_index.reshape(2, erows, K)

    deg_pp = _make_degree_kernel(npad, erows)(ei)

    blk = 1024
    dinv, xs = _make_tc_scale(n, npad, d_in, blk)(deg_pp, x)

    agg1 = _make_prop_kernel(npad, erows, d_in, tc_tiling=False)(xs, ei)

    w2p = jnp.pad(W2, ((0, 0), (0, d_out_p - d_out)))
    zs = _make_tc_mlp(npad, d_in, d_hid, d_out_p, blk)(
        agg1, xs, dinv,
        W1, b1.reshape(1, d_hid), gamma1.reshape(1, d_hid),
        beta1.reshape(1, d_hid), w2p)

    agg2 = _make_prop_kernel(npad, erows, d_out_p, tc_tiling=False)(zs, ei)

    b2p = jnp.pad(b2, (0, d_out_p - d_out)).reshape(1, d_out_p)
    return _make_tc_final(n, npad, d_out_p, d_out, 1000)(agg2, zs, dinv, b2p)
